# Initial kernel scaffold; baseline (speedup 1.0000x reference)
import jax
import jax.numpy as jnp
from jax.experimental import pallas as pl


def kernel(x, edge_index, t, W1, b1, mg, mb, W2, b2, lng, lnb):
    raise NotImplementedError("write your pallas kernel here")


# speedup vs baseline: 12.4859x; 12.4859x over previous
import functools

import jax
import jax.numpy as jnp
from jax import lax
from jax.experimental import pallas as pl
from jax.experimental.pallas import tpu as pltpu
from jax.experimental.pallas import tpu_sc as plsc

N = 10000
E = 320000
D = 128
H = 256
L = 4
EPS = 1e-7
SHIFT = 12.0
BR = 1000
NB = N // BR

CW = 125
ROWS_ALL = E // CW
ROWS_TILE = ROWS_ALL // 16
SUP = 16
N_PAD = 10240
DUMP = 624


def _ln(x, g, b, eps=1e-5):
    mu = jnp.mean(x, axis=-1, keepdims=True)
    var = jnp.mean((x - mu) ** 2, axis=-1, keepdims=True)
    return (x - mu) / jnp.sqrt(var + eps) * g + b


def _pre0_body(t_ref, x_ref, m_ref, cmax_ref):
    m = jnp.maximum(x_ref[...], 0.0) + EPS
    m_ref[...] = m
    lg = m * t_ref[0, 0]
    bm = jnp.max(lg, axis=0, keepdims=True)

    @pl.when(pl.program_id(0) == 0)
    def _():
        cmax_ref[...] = jnp.full((1, D), -jnp.inf, jnp.float32)

    cmax_ref[...] = jnp.maximum(cmax_ref[...], bm)


def _tab0_body(t_ref, m_ref, cmax_ref, pq_ref):
    m = m_ref[...]
    p = jnp.exp(m * t_ref[0, 0] - cmax_ref[...])
    pq_ref[0] = p
    pq_ref[1] = m * p


def _dense_body(t_ref, sums_ref, zin_ref, hprev_ref, w1_ref, b1_ref, mg_ref,
                mb_ref, w2_ref, b2_ref, g_ref, bb_ref, h_ref, *rest,
                has_resid, final):
    agg = sums_ref[1] / (sums_ref[0] + 1e-16)
    oc = agg + zin_ref[...]
    y = jnp.dot(oc, w1_ref[...], preferred_element_type=jnp.float32) + b1_ref[...]
    y = jnp.maximum(_ln(y, mg_ref[...], mb_ref[...]), 0.0)
    h = jnp.dot(y, w2_ref[...], preferred_element_type=jnp.float32) + b2_ref[...]
    if has_resid:
        h = h + hprev_ref[...]
    if final:
        h_ref[...] = jnp.maximum(_ln(h, g_ref[...], bb_ref[...]), 0.0)
    else:
        z_ref, pq_ref = rest
        h_ref[...] = h
        z = jnp.maximum(_ln(h, g_ref[...], bb_ref[...]), 0.0)
        z_ref[...] = z
        m = z + EPS
        p = jnp.exp(m * t_ref[0, 0] - SHIFT)
        pq_ref[0] = p
        pq_ref[1] = m * p


def _row_spec(i_map=None):
    return pl.BlockSpec((BR, D), i_map or (lambda i: (i, 0)))


def _full_spec(shape):
    return pl.BlockSpec(shape, lambda i: tuple(0 for _ in shape))


_SMEM_SPEC = pl.BlockSpec(memory_space=pltpu.SMEM)


def _pre0(x, t0):
    return pl.pallas_call(
        _pre0_body,
        grid=(NB,),
        in_specs=[_SMEM_SPEC, _row_spec()],
        out_specs=[_row_spec(), _full_spec((1, D))],
        out_shape=[jax.ShapeDtypeStruct((N, D), jnp.float32),
                   jax.ShapeDtypeStruct((1, D), jnp.float32)],
    )(t0, x)


def _tab0(m, cmax, t0):
    return pl.pallas_call(
        _tab0_body,
        grid=(NB,),
        in_specs=[_SMEM_SPEC, _row_spec(), _full_spec((1, D))],
        out_specs=pl.BlockSpec((2, BR, D), lambda i: (0, i, 0)),
        out_shape=jax.ShapeDtypeStruct((2, N, D), jnp.float32),
    )(t0, m, cmax)


def _dense(sums, zin, hprev, w1, b1, mg, mb, w2, b2, g, bb, tn,
           has_resid, final):
    body = functools.partial(_dense_body, has_resid=has_resid, final=final)
    out_specs = [_row_spec()]
    out_shape = [jax.ShapeDtypeStruct((N, D), jnp.float32)]
    if not final:
        out_specs += [_row_spec(), pl.BlockSpec((2, BR, D), lambda i: (0, i, 0))]
        out_shape += [jax.ShapeDtypeStruct((N, D), jnp.float32),
                      jax.ShapeDtypeStruct((2, N, D), jnp.float32)]
    return pl.pallas_call(
        body,
        grid=(NB,),
        in_specs=[_SMEM_SPEC,
                  pl.BlockSpec((2, BR, D), lambda i: (0, i, 0)),
                  _row_spec(),
                  _row_spec(),
                  _full_spec((D, H)), _full_spec((1, H)),
                  _full_spec((1, H)), _full_spec((1, H)),
                  _full_spec((H, D)), _full_spec((1, D)),
                  _full_spec((1, D)), _full_spec((1, D))],
        out_specs=out_specs,
        out_shape=out_shape,
    )(tn, sums, zin, hprev, w1, b1, mg, mb, w2, b2, g, bb)


def _sc_body(tab, srcb, dstb, out, sbuf, dbuf, rows, zbuf, acc, sem):
    c = lax.axis_index("c")
    s = lax.axis_index("s")

    def _zrow(r, _):
        for k in range(D // 16):
            zbuf[r, pl.ds(k * 16, 16)] = jnp.zeros((16,), jnp.float32)
        return 0

    lax.fori_loop(0, 128, _zrow, 0)
    for j in range(5):
        pltpu.sync_copy(zbuf, acc.at[pl.ds(pl.multiple_of(s * 640, 8) + j * 128, 128)])
    plsc.subcore_barrier()

    def _outer(g, _):
        row0 = pl.multiple_of(s * ROWS_TILE + g * SUP, 8)
        pltpu.sync_copy(srcb.at[pl.ds(pl.multiple_of(c * ROWS_ALL, 8) + row0, SUP)], sbuf)
        pltpu.sync_copy(dstb.at[pl.ds(row0, SUP)], dbuf)

        def _inner(j, _):
            pltpu.async_copy(tab.at[sbuf.at[j]], rows, sem).wait()
            pltpu.sync_copy(rows, acc.at[dbuf.at[j]], add=True)
            return 0

        lax.fori_loop(0, SUP, _inner, 0)
        return 0

    lax.fori_loop(0, ROWS_TILE // SUP, _outer, 0)
    plsc.subcore_barrier()

    pltpu.sync_copy(acc.at[pl.ds(pl.multiple_of(s * DUMP, 8), DUMP)],
                    out.at[pl.ds(pl.multiple_of(c * N + s * DUMP, 8), DUMP)])

    @pl.when(s == 15)
    def _():
        pltpu.sync_copy(acc.at[pl.ds(16 * DUMP, 16)],
                        out.at[pl.ds(pl.multiple_of(c * N, 8) + 16 * DUMP, 16)])


def _sc_scatter(tab, srcb, dstb):
    mesh = plsc.VectorSubcoreMesh(core_axis_name="c", subcore_axis_name="s")
    return pl.kernel(
        _sc_body,
        mesh=mesh,
        out_type=jax.ShapeDtypeStruct((2 * N, D), jnp.float32),
        scratch_types=[
            pltpu.VMEM((SUP, CW), jnp.int32),
            pltpu.VMEM((SUP, CW), jnp.int32),
            pltpu.VMEM((CW, D), jnp.float32),
            pltpu.VMEM((128, D), jnp.float32),
            pltpu.VMEM_SHARED((N_PAD, D), jnp.float32),
            pltpu.SemaphoreType.DMA,
        ],
    )(tab, srcb, dstb)


def kernel(x, edge_index, t, W1, b1, mg, mb, W2, b2, lng, lnb):
    src = edge_index[0]
    dst = edge_index[1]
    srcb = jnp.concatenate([src, src + N]).reshape(2 * ROWS_ALL, CW)
    dstb = dst.reshape(ROWS_ALL, CW)
    ts = t.reshape(L, 1, 1)

    m0, cmax = _pre0(x, ts[0])
    pq = _tab0(m0, cmax, ts[0])

    h = None
    zin = x
    for l in range(L):
        sums = _sc_scatter(pq.reshape(2 * N, D), srcb, dstb).reshape(2, N, D)
        final = l == L - 1
        g_idx = 0 if final else l + 1
        tn = ts[0] if final else ts[l + 1]
        outs = _dense(sums, zin, x if h is None else h,
                      W1[l], b1[l].reshape(1, H), mg[l].reshape(1, H),
                      mb[l].reshape(1, H), W2[l], b2[l].reshape(1, D),
                      lng[g_idx].reshape(1, D), lnb[g_idx].reshape(1, D),
                      tn, has_resid=l > 0, final=final)
        if final:
            return outs[0]
        h, zin, pq = outs


# speedup vs baseline: 16.6885x; 1.3366x over previous
import functools

import jax
import jax.numpy as jnp
from jax import lax
from jax.experimental import pallas as pl
from jax.experimental.pallas import tpu as pltpu
from jax.experimental.pallas import tpu_sc as plsc

N = 10000
E = 320000
D = 128
H = 256
L = 4
EPS = 1e-7
SHIFT = 12.0
BR = 1000
NB = N // BR

CW = 125
ROWS_ALL = E // CW
ROWS_TILE = ROWS_ALL // 16
SUP = 32
N_PAD = 10240
DUMP = 624


def _ln(x, g, b, eps=1e-5):
    mu = jnp.mean(x, axis=-1, keepdims=True)
    var = jnp.mean((x - mu) ** 2, axis=-1, keepdims=True)
    return (x - mu) / jnp.sqrt(var + eps) * g + b


def _pre0_body(t_ref, x_ref, m_ref, cmax_ref):
    m = jnp.maximum(x_ref[...], 0.0) + EPS
    m_ref[...] = m
    lg = m * t_ref[0, 0]
    bm = jnp.max(lg, axis=0, keepdims=True)

    @pl.when(pl.program_id(0) == 0)
    def _():
        cmax_ref[...] = jnp.full((1, D), -jnp.inf, jnp.float32)

    cmax_ref[...] = jnp.maximum(cmax_ref[...], bm)


def _tab0_body(t_ref, m_ref, cmax_ref, pq_ref):
    m = m_ref[...]
    p = jnp.exp(m * t_ref[0, 0] - cmax_ref[...])
    pq_ref[0] = p
    pq_ref[1] = m * p


def _dense_body(t_ref, sums_ref, zin_ref, hprev_ref, w1_ref, b1_ref, mg_ref,
                mb_ref, w2_ref, b2_ref, g_ref, bb_ref, h_ref, *rest,
                has_resid, final):
    agg = sums_ref[1] / (sums_ref[0] + 1e-16)
    oc = agg + zin_ref[...]
    y = jnp.dot(oc, w1_ref[...], preferred_element_type=jnp.float32) + b1_ref[...]
    y = jnp.maximum(_ln(y, mg_ref[...], mb_ref[...]), 0.0)
    h = jnp.dot(y, w2_ref[...], preferred_element_type=jnp.float32) + b2_ref[...]
    if has_resid:
        h = h + hprev_ref[...]
    if final:
        h_ref[...] = jnp.maximum(_ln(h, g_ref[...], bb_ref[...]), 0.0)
    else:
        z_ref, pq_ref = rest
        h_ref[...] = h
        z = jnp.maximum(_ln(h, g_ref[...], bb_ref[...]), 0.0)
        z_ref[...] = z
        m = z + EPS
        p = jnp.exp(m * t_ref[0, 0] - SHIFT)
        pq_ref[0] = p
        pq_ref[1] = m * p


def _row_spec(i_map=None):
    return pl.BlockSpec((BR, D), i_map or (lambda i: (i, 0)))


def _full_spec(shape):
    return pl.BlockSpec(shape, lambda i: tuple(0 for _ in shape))


_SMEM_SPEC = pl.BlockSpec(memory_space=pltpu.SMEM)


def _pre0(x, t0):
    return pl.pallas_call(
        _pre0_body,
        grid=(NB,),
        in_specs=[_SMEM_SPEC, _row_spec()],
        out_specs=[_row_spec(), _full_spec((1, D))],
        out_shape=[jax.ShapeDtypeStruct((N, D), jnp.float32),
                   jax.ShapeDtypeStruct((1, D), jnp.float32)],
    )(t0, x)


def _tab0(m, cmax, t0):
    return pl.pallas_call(
        _tab0_body,
        grid=(NB,),
        in_specs=[_SMEM_SPEC, _row_spec(), _full_spec((1, D))],
        out_specs=pl.BlockSpec((2, BR, D), lambda i: (0, i, 0)),
        out_shape=jax.ShapeDtypeStruct((2, N, D), jnp.float32),
    )(t0, m, cmax)


def _dense(sums, zin, hprev, w1, b1, mg, mb, w2, b2, g, bb, tn,
           has_resid, final):
    body = functools.partial(_dense_body, has_resid=has_resid, final=final)
    out_specs = [_row_spec()]
    out_shape = [jax.ShapeDtypeStruct((N, D), jnp.float32)]
    if not final:
        out_specs += [_row_spec(), pl.BlockSpec((2, BR, D), lambda i: (0, i, 0))]
        out_shape += [jax.ShapeDtypeStruct((N, D), jnp.float32),
                      jax.ShapeDtypeStruct((2, N, D), jnp.float32)]
    return pl.pallas_call(
        body,
        grid=(NB,),
        in_specs=[_SMEM_SPEC,
                  pl.BlockSpec((2, BR, D), lambda i: (0, i, 0)),
                  _row_spec(),
                  _row_spec(),
                  _full_spec((D, H)), _full_spec((1, H)),
                  _full_spec((1, H)), _full_spec((1, H)),
                  _full_spec((H, D)), _full_spec((1, D)),
                  _full_spec((1, D)), _full_spec((1, D))],
        out_specs=out_specs,
        out_shape=out_shape,
    )(tn, sums, zin, hprev, w1, b1, mg, mb, w2, b2, g, bb)


def _sc_body(tab, srcb, dstb, out, sbuf, dbuf, rows0, rows1, zbuf, acc,
             gsem0, gsem1, ssem0, ssem1):
    c = lax.axis_index("c")
    s = lax.axis_index("s")

    def _zrow(r, _):
        for k in range(D // 16):
            zbuf[r, pl.ds(k * 16, 16)] = jnp.zeros((16,), jnp.float32)
        return 0

    lax.fori_loop(0, 16, _zrow, 0)

    def _zcp(j, _):
        pltpu.sync_copy(zbuf, acc.at[pl.ds(pl.multiple_of(s * 640, 8) + j * 16, 16)])
        return 0

    lax.fori_loop(0, 40, _zcp, 0)
    plsc.subcore_barrier()

    ng = SUP // 2

    def _stage(st, _):
        row0 = pl.multiple_of(s * ROWS_TILE + st * SUP, 8)
        pltpu.sync_copy(srcb.at[pl.ds(pl.multiple_of(c * ROWS_ALL, 8) + row0, SUP)], sbuf)
        pltpu.sync_copy(dstb.at[pl.ds(row0, SUP)], dbuf)
        pltpu.async_copy(tab.at[sbuf.at[0]], rows0, gsem0)

        def _step(g, _):
            i0 = 2 * g
            pltpu.make_async_copy(tab.at[sbuf.at[i0]], rows0, gsem0).wait()

            @pl.when(g > 0)
            def _():
                pltpu.make_async_copy(rows1, acc.at[dbuf.at[i0 - 1]], ssem1).wait()

            pltpu.async_copy(tab.at[sbuf.at[i0 + 1]], rows1, gsem1)
            pltpu.async_copy(rows0, acc.at[dbuf.at[i0]], ssem0, add=True)
            pltpu.make_async_copy(tab.at[sbuf.at[i0 + 1]], rows1, gsem1).wait()

            @pl.when(g < ng - 1)
            def _():
                pltpu.make_async_copy(rows0, acc.at[dbuf.at[i0]], ssem0).wait()
                pltpu.async_copy(tab.at[sbuf.at[i0 + 2]], rows0, gsem0)

            pltpu.async_copy(rows1, acc.at[dbuf.at[i0 + 1]], ssem1, add=True)
            return 0

        lax.fori_loop(0, ng, _step, 0)
        pltpu.make_async_copy(rows0, acc.at[dbuf.at[SUP - 2]], ssem0).wait()
        pltpu.make_async_copy(rows1, acc.at[dbuf.at[SUP - 1]], ssem1).wait()
        return 0

    lax.fori_loop(0, ROWS_TILE // SUP, _stage, 0)
    plsc.subcore_barrier()

    pltpu.sync_copy(acc.at[pl.ds(pl.multiple_of(s * DUMP, 8), DUMP)],
                    out.at[pl.ds(pl.multiple_of(c * N + s * DUMP, 8), DUMP)])

    @pl.when(s == 15)
    def _():
        pltpu.sync_copy(acc.at[pl.ds(16 * DUMP, 16)],
                        out.at[pl.ds(pl.multiple_of(c * N, 8) + 16 * DUMP, 16)])


def _sc_scatter(tab, srcb, dstb):
    mesh = plsc.VectorSubcoreMesh(core_axis_name="c", subcore_axis_name="s")
    return pl.kernel(
        _sc_body,
        mesh=mesh,
        out_type=jax.ShapeDtypeStruct((2 * N, D), jnp.float32),
        scratch_types=[
            pltpu.VMEM((SUP, CW), jnp.int32),
            pltpu.VMEM((SUP, CW), jnp.int32),
            pltpu.VMEM((CW, D), jnp.float32),
            pltpu.VMEM((CW, D), jnp.float32),
            pltpu.VMEM((16, D), jnp.float32),
            pltpu.VMEM_SHARED((N_PAD, D), jnp.float32),
            pltpu.SemaphoreType.DMA,
            pltpu.SemaphoreType.DMA,
            pltpu.SemaphoreType.DMA,
            pltpu.SemaphoreType.DMA,
        ],
    )(tab, srcb, dstb)


def kernel(x, edge_index, t, W1, b1, mg, mb, W2, b2, lng, lnb):
    src = edge_index[0]
    dst = edge_index[1]
    srcb = jnp.concatenate([src, src + N]).reshape(2 * ROWS_ALL, CW)
    dstb = dst.reshape(ROWS_ALL, CW)
    ts = t.reshape(L, 1, 1)

    m0, cmax = _pre0(x, ts[0])
    pq = _tab0(m0, cmax, ts[0])

    h = None
    zin = x
    for l in range(L):
        sums = _sc_scatter(pq.reshape(2 * N, D), srcb, dstb).reshape(2, N, D)
        final = l == L - 1
        g_idx = 0 if final else l + 1
        tn = ts[0] if final else ts[l + 1]
        outs = _dense(sums, zin, x if h is None else h,
                      W1[l], b1[l].reshape(1, H), mg[l].reshape(1, H),
                      mb[l].reshape(1, H), W2[l], b2[l].reshape(1, D),
                      lng[g_idx].reshape(1, D), lnb[g_idx].reshape(1, D),
                      tn, has_resid=l > 0, final=final)
        if final:
            return outs[0]
        h, zin, pq = outs


# speedup vs baseline: 20.3534x; 1.2196x over previous
import functools

import jax
import jax.numpy as jnp
from jax import lax
from jax.experimental import pallas as pl
from jax.experimental.pallas import tpu as pltpu
from jax.experimental.pallas import tpu_sc as plsc

N = 10000
E = 320000
D = 128
H = 256
L = 4
EPS = 1e-7
SHIFT = 12.0
BR = 1000
NB = N // BR

CW = 50
ROWS_ALL = E // CW
ROWS_TILE = ROWS_ALL // 16
SUP = 40
W = 4
RING = SUP // W
N_PAD = 10240
DUMP = 624


def _ln(x, g, b, eps=1e-5):
    mu = jnp.mean(x, axis=-1, keepdims=True)
    var = jnp.mean((x - mu) ** 2, axis=-1, keepdims=True)
    return (x - mu) / jnp.sqrt(var + eps) * g + b


def _pre0_body(t_ref, x_ref, m_ref, cmax_ref):
    m = jnp.maximum(x_ref[...], 0.0) + EPS
    m_ref[...] = m
    lg = m * t_ref[0, 0]
    bm = jnp.max(lg, axis=0, keepdims=True)

    @pl.when(pl.program_id(0) == 0)
    def _():
        cmax_ref[...] = jnp.full((1, D), -jnp.inf, jnp.float32)

    cmax_ref[...] = jnp.maximum(cmax_ref[...], bm)


def _tab0_body(t_ref, m_ref, cmax_ref, pq_ref):
    m = m_ref[...]
    p = jnp.exp(m * t_ref[0, 0] - cmax_ref[...])
    pq_ref[0] = p
    pq_ref[1] = m * p


def _dense_body(t_ref, sums_ref, zin_ref, hprev_ref, w1_ref, b1_ref, mg_ref,
                mb_ref, w2_ref, b2_ref, g_ref, bb_ref, h_ref, *rest,
                has_resid, final):
    agg = sums_ref[1] / (sums_ref[0] + 1e-16)
    oc = agg + zin_ref[...]
    y = jnp.dot(oc, w1_ref[...], preferred_element_type=jnp.float32) + b1_ref[...]
    y = jnp.maximum(_ln(y, mg_ref[...], mb_ref[...]), 0.0)
    h = jnp.dot(y, w2_ref[...], preferred_element_type=jnp.float32) + b2_ref[...]
    if has_resid:
        h = h + hprev_ref[...]
    if final:
        h_ref[...] = jnp.maximum(_ln(h, g_ref[...], bb_ref[...]), 0.0)
    else:
        z_ref, pq_ref = rest
        h_ref[...] = h
        z = jnp.maximum(_ln(h, g_ref[...], bb_ref[...]), 0.0)
        z_ref[...] = z
        m = z + EPS
        p = jnp.exp(m * t_ref[0, 0] - SHIFT)
        pq_ref[0] = p
        pq_ref[1] = m * p


def _row_spec(i_map=None):
    return pl.BlockSpec((BR, D), i_map or (lambda i: (i, 0)))


def _full_spec(shape):
    return pl.BlockSpec(shape, lambda i: tuple(0 for _ in shape))


_SMEM_SPEC = pl.BlockSpec(memory_space=pltpu.SMEM)


def _pre0(x, t0):
    return pl.pallas_call(
        _pre0_body,
        grid=(NB,),
        in_specs=[_SMEM_SPEC, _row_spec()],
        out_specs=[_row_spec(), _full_spec((1, D))],
        out_shape=[jax.ShapeDtypeStruct((N, D), jnp.float32),
                   jax.ShapeDtypeStruct((1, D), jnp.float32)],
    )(t0, x)


def _tab0(m, cmax, t0):
    return pl.pallas_call(
        _tab0_body,
        grid=(NB,),
        in_specs=[_SMEM_SPEC, _row_spec(), _full_spec((1, D))],
        out_specs=pl.BlockSpec((2, BR, D), lambda i: (0, i, 0)),
        out_shape=jax.ShapeDtypeStruct((2, N, D), jnp.float32),
    )(t0, m, cmax)


def _dense(sums, zin, hprev, w1, b1, mg, mb, w2, b2, g, bb, tn,
           has_resid, final):
    body = functools.partial(_dense_body, has_resid=has_resid, final=final)
    out_specs = [_row_spec()]
    out_shape = [jax.ShapeDtypeStruct((N, D), jnp.float32)]
    if not final:
        out_specs += [_row_spec(), pl.BlockSpec((2, BR, D), lambda i: (0, i, 0))]
        out_shape += [jax.ShapeDtypeStruct((N, D), jnp.float32),
                      jax.ShapeDtypeStruct((2, N, D), jnp.float32)]
    return pl.pallas_call(
        body,
        grid=(NB,),
        in_specs=[_SMEM_SPEC,
                  pl.BlockSpec((2, BR, D), lambda i: (0, i, 0)),
                  _row_spec(),
                  _row_spec(),
                  _full_spec((D, H)), _full_spec((1, H)),
                  _full_spec((1, H)), _full_spec((1, H)),
                  _full_spec((H, D)), _full_spec((1, D)),
                  _full_spec((1, D)), _full_spec((1, D))],
        out_specs=out_specs,
        out_shape=out_shape,
    )(tn, sums, zin, hprev, w1, b1, mg, mb, w2, b2, g, bb)


def _sc_body(tab, srcb, dstb, out, sbuf, dbuf, r0, r1, r2, r3, acc,
             g0, g1, g2, g3, s0, s1, s2, s3):
    rows = [r0, r1, r2, r3]
    gs = [g0, g1, g2, g3]
    ss = [s0, s1, s2, s3]
    c = lax.axis_index("c")
    s = lax.axis_index("s")

    def _zrow(r, _):
        for k in range(D // 16):
            r0[r, pl.ds(k * 16, 16)] = jnp.zeros((16,), jnp.float32)
        return 0

    lax.fori_loop(0, 40, _zrow, 0)

    def _zcp(j, _):
        pltpu.sync_copy(r0.at[pl.ds(0, 40)],
                        acc.at[pl.ds(pl.multiple_of(s * 640, 8) + j * 40, 40)])
        return 0

    lax.fori_loop(0, 16, _zcp, 0)
    plsc.subcore_barrier()

    def _stage(st, _):
        row0 = pl.multiple_of(s * ROWS_TILE + st * SUP, 8)
        pltpu.sync_copy(srcb.at[pl.ds(pl.multiple_of(c * ROWS_ALL, 8) + row0, SUP)], sbuf)
        pltpu.sync_copy(dstb.at[pl.ds(row0, SUP)], dbuf)
        for b in range(W):
            pltpu.async_copy(tab.at[sbuf.at[b]], rows[b], gs[b])

        def _ring(g, _):
            for b in range(W):
                i = W * g + b
                pltpu.make_async_copy(tab.at[sbuf.at[i]], rows[b], gs[b]).wait()
                pltpu.async_copy(rows[b], acc.at[dbuf.at[i]], ss[b], add=True)

                @pl.when(g < RING - 1)
                def _():
                    pltpu.make_async_copy(rows[b], acc.at[dbuf.at[i]], ss[b]).wait()
                    pltpu.async_copy(tab.at[sbuf.at[i + W]], rows[b], gs[b])
            return 0

        lax.fori_loop(0, RING, _ring, 0)
        for b in range(W):
            pltpu.make_async_copy(rows[b], acc.at[dbuf.at[SUP - W + b]], ss[b]).wait()
        return 0

    lax.fori_loop(0, ROWS_TILE // SUP, _stage, 0)
    plsc.subcore_barrier()

    pltpu.sync_copy(acc.at[pl.ds(pl.multiple_of(s * DUMP, 8), DUMP)],
                    out.at[pl.ds(pl.multiple_of(c * N + s * DUMP, 8), DUMP)])

    @pl.when(s == 15)
    def _():
        pltpu.sync_copy(acc.at[pl.ds(16 * DUMP, 16)],
                        out.at[pl.ds(pl.multiple_of(c * N, 8) + 16 * DUMP, 16)])


def _sc_scatter(tab, srcb, dstb):
    mesh = plsc.VectorSubcoreMesh(core_axis_name="c", subcore_axis_name="s")
    return pl.kernel(
        _sc_body,
        mesh=mesh,
        out_type=jax.ShapeDtypeStruct((2 * N, D), jnp.float32),
        scratch_types=(
            [pltpu.VMEM((SUP, CW), jnp.int32)] * 2
            + [pltpu.VMEM((CW, D), jnp.float32)] * W
            + [pltpu.VMEM_SHARED((N_PAD, D), jnp.float32)]
            + [pltpu.SemaphoreType.DMA] * (2 * W)
        ),
    )(tab, srcb, dstb)


def kernel(x, edge_index, t, W1, b1, mg, mb, W2, b2, lng, lnb):
    src = edge_index[0]
    dst = edge_index[1]
    srcb = jnp.concatenate([src, src + N]).reshape(2 * ROWS_ALL, CW)
    dstb = dst.reshape(ROWS_ALL, CW)
    ts = t.reshape(L, 1, 1)

    m0, cmax = _pre0(x, ts[0])
    pq = _tab0(m0, cmax, ts[0])

    h = None
    zin = x
    for l in range(L):
        sums = _sc_scatter(pq.reshape(2 * N, D), srcb, dstb).reshape(2, N, D)
        final = l == L - 1
        g_idx = 0 if final else l + 1
        tn = ts[0] if final else ts[l + 1]
        outs = _dense(sums, zin, x if h is None else h,
                      W1[l], b1[l].reshape(1, H), mg[l].reshape(1, H),
                      mb[l].reshape(1, H), W2[l], b2[l].reshape(1, D),
                      lng[g_idx].reshape(1, D), lnb[g_idx].reshape(1, D),
                      tn, has_resid=l > 0, final=final)
        if final:
            return outs[0]
        h, zin, pq = outs
